# baseline (speedup 1.0000x reference)
import functools

import jax
import jax.numpy as jnp
from jax import lax
from jax.experimental import pallas as pl
from jax.experimental.pallas import tpu as pltpu
from jax.experimental.pallas import tpu_sc as plsc

N_TOKENS = 32768
D_MODEL = 768
N_EXPERTS = 64
CAPACITY = 512
GATE_H = 64
EXP_H = 128
FLAT_ROWS = N_EXPERTS * CAPACITY + 1

TB = 1024
N_TBLOCKS = N_TOKENS // TB
D_HALF = D_MODEL // 2


def _pack_bf16(x32):
    lo = lax.bitcast_convert_type(x32[:, :D_HALF].astype(jnp.bfloat16),
                                  jnp.uint16).astype(jnp.uint32)
    hi = lax.bitcast_convert_type(x32[:, D_HALF:].astype(jnp.bfloat16),
                                  jnp.uint16).astype(jnp.uint32)
    return lax.bitcast_convert_type(lo | (hi << 16), jnp.float32)


def _unpack_bf16(p32):
    u = lax.bitcast_convert_type(p32, jnp.uint32)
    lo = lax.bitcast_convert_type((u & 0xFFFF).astype(jnp.uint16),
                                  jnp.bfloat16)
    hi = lax.bitcast_convert_type((u >> 16).astype(jnp.uint16), jnp.bfloat16)
    return jnp.concatenate([lo, hi], axis=1)

NC, NS = 2, 16
NW = NC * NS
TOK_PER_W = N_TOKENS // NW
K_CHUNK = 64
N_CHUNKS = TOK_PER_W // K_CHUNK
N_BUF = 4
N_ROUNDS = N_CHUNKS // N_BUF


def _gate_body(x_ref, wg1_ref, bg1_ref, wg2_ref, bg2_ref, sslot_ref, stok_ref,
               keep_ref, kcnt_ref, xbf_ref, counts_ref):
    i = pl.program_id(0)

    @pl.when(i == 0)
    def _():
        counts_ref[...] = jnp.zeros_like(counts_ref)

    x = x_ref[...]
    xbf_ref[...] = _pack_bf16(x)
    gh = jnp.dot(x, wg1_ref[...], preferred_element_type=jnp.float32)
    gh = jnp.maximum(gh + bg1_ref[...], 0.0)
    logits = jnp.dot(gh, wg2_ref[...], preferred_element_type=jnp.float32)
    logits = logits + bg2_ref[...]

    m = jnp.max(logits, axis=1, keepdims=True)
    iota_e = lax.broadcasted_iota(jnp.int32, (TB, N_EXPERTS), 1)
    sel = logits == m
    expert = jnp.min(jnp.where(sel, iota_e, N_EXPERTS), axis=1)
    first = expert[:, None] == iota_e

    ii = lax.broadcasted_iota(jnp.int32, (TB, TB), 0)
    jj = lax.broadcasted_iota(jnp.int32, (TB, TB), 1)
    tri = (jj <= ii).astype(jnp.bfloat16)
    onehot = first.astype(jnp.bfloat16)
    pos_incl = jnp.dot(tri, onehot, preferred_element_type=jnp.float32)

    prev = counts_ref[...]
    pos_all = pos_incl + prev
    pos_tok = jnp.sum(jnp.where(first, pos_all, 0.0), axis=1) - 1.0
    keep = pos_tok < CAPACITY
    slot = jnp.where(
        keep,
        expert * CAPACITY + pos_tok.astype(jnp.int32),
        N_EXPERTS * CAPACITY,
    )
    counts_ref[...] = prev + jnp.sum(first.astype(jnp.float32), axis=0,
                                     keepdims=True)

    keep_f = keep.astype(jnp.float32)
    kept_first = jnp.where(first, keep_f[:, None], 0.0)
    kcnt = jnp.sum(kept_first, axis=0, keepdims=True)
    e_a = lax.broadcasted_iota(jnp.int32, (N_EXPERTS, N_EXPERTS), 0)
    e_b = lax.broadcasted_iota(jnp.int32, (N_EXPERTS, N_EXPERTS), 1)
    t64 = (e_a < e_b).astype(jnp.bfloat16)
    prefix_e = (jnp.dot((kcnt // 256).astype(jnp.bfloat16), t64,
                        preferred_element_type=jnp.float32) * 256.0
                + jnp.dot((kcnt % 256).astype(jnp.bfloat16), t64,
                          preferred_element_type=jnp.float32))
    own_pos = jnp.sum(jnp.where(first, pos_incl, 0.0), axis=1) - 1.0
    pref_at_e = jnp.sum(jnp.where(first, prefix_e, 0.0), axis=1)
    drop_bf = (1.0 - keep_f).astype(jnp.bfloat16)
    dincl = jnp.dot(tri, drop_bf[:, None],
                    preferred_element_type=jnp.float32)[:, 0]
    total_kept = jnp.sum(keep_f)
    rank = jnp.where(keep, pref_at_e + own_pos,
                     total_kept + dincl - 1.0).astype(jnp.int32)

    CT = 256
    li = lax.broadcasted_iota(jnp.int32, (TB,), 0)
    vals = jnp.concatenate([
        (slot // 256).astype(jnp.bfloat16)[None, :],
        (slot % 256).astype(jnp.bfloat16)[None, :],
        (li // 256).astype(jnp.bfloat16)[None, :],
        (li % 256).astype(jnp.bfloat16)[None, :],
    ], axis=0)
    srt_parts = []
    for t in range(TB // CT):
        cj = lax.broadcasted_iota(jnp.int32, (TB, CT), 1) + t * CT
        pt = (rank[:, None] == cj).astype(jnp.bfloat16)
        srt_parts.append(jnp.dot(vals, pt, preferred_element_type=jnp.float32))
    srt = jnp.concatenate(srt_parts, axis=1)
    sslot_ref[...] = (srt[0] * 256.0 + srt[1]).astype(jnp.int32)[None, None, :]
    stok_ref[...] = ((srt[2] * 256.0 + srt[3]).astype(jnp.int32)
                     + i * TB)[None, None, :]
    keep_ref[...] = keep.astype(jnp.int32)[None, None, :]
    kcnt_ref[...] = jnp.full((1, 1, 128), total_kept.astype(jnp.int32),
                             jnp.int32)


def _gate_route(x, wg1, bg1, wg2, bg2):
    return pl.pallas_call(
        _gate_body,
        grid=(N_TBLOCKS,),
        in_specs=[
            pl.BlockSpec((TB, D_MODEL), lambda i: (i, 0)),
            pl.BlockSpec((D_MODEL, GATE_H), lambda i: (0, 0)),
            pl.BlockSpec((1, GATE_H), lambda i: (0, 0)),
            pl.BlockSpec((GATE_H, N_EXPERTS), lambda i: (0, 0)),
            pl.BlockSpec((1, N_EXPERTS), lambda i: (0, 0)),
        ],
        out_specs=[
            pl.BlockSpec((1, 1, TB), lambda i: (i, 0, 0)),
            pl.BlockSpec((1, 1, TB), lambda i: (i, 0, 0)),
            pl.BlockSpec((1, 1, TB), lambda i: (i, 0, 0)),
            pl.BlockSpec((1, 1, 128), lambda i: (i, 0, 0)),
            pl.BlockSpec((TB, D_HALF), lambda i: (i, 0)),
        ],
        out_shape=[
            jax.ShapeDtypeStruct((N_TBLOCKS, 1, TB), jnp.int32),
            jax.ShapeDtypeStruct((N_TBLOCKS, 1, TB), jnp.int32),
            jax.ShapeDtypeStruct((N_TBLOCKS, 1, TB), jnp.int32),
            jax.ShapeDtypeStruct((N_TBLOCKS, 1, 128), jnp.int32),
            jax.ShapeDtypeStruct((N_TOKENS, D_HALF), jnp.float32),
        ],
        scratch_shapes=[pltpu.VMEM((1, N_EXPERTS), jnp.float32)],
    )(x, wg1, bg1, wg2, bg2)


def _expert_body(disp_ref, w1_ref, b1_ref, w2_ref, b2_ref, out_ref):
    e = pl.program_id(0)

    @pl.when(e < N_EXPERTS)
    def _():
        xb = _unpack_bf16(disp_ref[...])
        h = jnp.dot(xb, w1_ref[0].astype(jnp.bfloat16),
                    preferred_element_type=jnp.float32)
        h = jnp.maximum(h + b1_ref[0], 0.0)
        z = jnp.dot(h.astype(jnp.bfloat16), w2_ref[0].astype(jnp.bfloat16),
                    preferred_element_type=jnp.float32)
        z = z + b2_ref[0]
        out_ref[...] = _pack_bf16(jax.nn.softmax(z, axis=-1))

    @pl.when(e == N_EXPERTS)
    def _():
        out_ref[...] = jnp.zeros_like(out_ref)


def _experts(disp, w1, b1, w2, b2):
    clamp = lambda e: jnp.minimum(e, N_EXPERTS - 1)
    return pl.pallas_call(
        _expert_body,
        grid=(N_EXPERTS + 1,),
        in_specs=[
            pl.BlockSpec((CAPACITY, D_HALF), lambda e: (clamp(e), 0)),
            pl.BlockSpec((1, D_MODEL, EXP_H), lambda e: (clamp(e), 0, 0)),
            pl.BlockSpec((1, 1, EXP_H), lambda e: (clamp(e), 0, 0)),
            pl.BlockSpec((1, EXP_H, D_MODEL), lambda e: (clamp(e), 0, 0)),
            pl.BlockSpec((1, 1, D_MODEL), lambda e: (clamp(e), 0, 0)),
        ],
        out_specs=pl.BlockSpec((CAPACITY, D_HALF), lambda e: (e, 0)),
        out_shape=jax.ShapeDtypeStruct((FLAT_ROWS, D_HALF), jnp.float32),
    )(disp, w1, b1.reshape(N_EXPERTS, 1, EXP_H), w2,
      b2.reshape(N_EXPERTS, 1, D_MODEL))


def _worker_id():
    return lax.axis_index("s") * NC + lax.axis_index("c")


@functools.cache
def _sc_kernels():
    mesh = plsc.VectorSubcoreMesh(core_axis_name="c", subcore_axis_name="s",
                                  num_cores=NC, num_subcores=NS)
    scratch = [
        pltpu.VMEM((N_CHUNKS, K_CHUNK), jnp.int32),
        pltpu.VMEM((N_CHUNKS, K_CHUNK), jnp.int32),
        pltpu.VMEM((K_CHUNK, D_HALF), jnp.float32),
        pltpu.SemaphoreType.DMA,
    ]

    def make_permute(n_out_rows):
        @functools.partial(
            pl.kernel,
            mesh=mesh,
            out_type=jax.ShapeDtypeStruct((n_out_rows, D_HALF), jnp.float32),
            scratch_types=scratch,
        )
        def permute(src_hbm, sidx_hbm, didx_hbm, out_hbm,
                    sidx_v, didx_v, rows_v, sem):
            wid = _worker_id()
            pltpu.sync_copy(sidx_hbm.at[wid], sidx_v)
            pltpu.sync_copy(didx_hbm.at[wid], didx_v)
            for k in range(N_CHUNKS):
                pltpu.async_copy(src_hbm.at[sidx_v.at[k]], rows_v, sem).wait()
                pltpu.async_copy(rows_v, out_hbm.at[didx_v.at[k]], sem).wait()

        return permute

    return make_permute(FLAT_ROWS), make_permute(N_TOKENS + 1)


def _unpack_body(p_ref, keep_ref, out_ref):
    full = _unpack_bf16(p_ref[...]).astype(jnp.float32)
    out_ref[...] = jnp.where(keep_ref[0, 0][:, None] > 0, full, 0.0)


def _final_unpack(packed, keep):
    return pl.pallas_call(
        _unpack_body,
        grid=(N_TBLOCKS,),
        in_specs=[
            pl.BlockSpec((TB, D_HALF), lambda i: (i, 0)),
            pl.BlockSpec((1, 1, TB), lambda i: (i, 0, 0)),
        ],
        out_specs=pl.BlockSpec((TB, D_MODEL), lambda i: (i, 0)),
        out_shape=jax.ShapeDtypeStruct((N_TOKENS, D_MODEL), jnp.float32),
    )(packed, keep)


def _interleave(a3):
    return (a3.reshape(16, 2, N_CHUNKS, K_CHUNK)
            .transpose(1, 2, 0, 3).reshape(NW, N_CHUNKS, K_CHUNK))


def kernel(inputs, Wg1, bg1, Wg2, bg2, W1, b1, W2, b2):
    sslot, stok, keep, kcnt, xbf = _gate_route(
        inputs, Wg1, bg1.reshape(1, -1), Wg2, bg2.reshape(1, -1))
    sslotW = _interleave(sslot.reshape(N_TBLOCKS, N_CHUNKS, K_CHUNK))
    stokW = _interleave(stok.reshape(N_TBLOCKS, N_CHUNKS, K_CHUNK))
    cnt = kcnt.reshape(N_TBLOCKS, 128)[:, 0]
    w = jnp.arange(NW)[:, None]
    k = jnp.arange(N_CHUNKS)[None, :]
    live = ((w % 16) * K_CHUNK < cnt[2 * k + w // 16])[:, :, None]
    d_src = jnp.where(live, stokW, 0)
    d_dst = jnp.where(live, sslotW, FLAT_ROWS - 1)
    c_src = jnp.where(live, sslotW, FLAT_ROWS - 1)
    c_dst = jnp.where(live, stokW, N_TOKENS)
    dispatch, combine = _sc_kernels()
    disp = dispatch(xbf, d_src, d_dst)
    flat = _experts(disp, W1, b1, W2, b2)
    return _final_unpack(combine(flat, c_src, c_dst), keep)


# speedup vs baseline: 4.3046x; 4.3046x over previous
import functools

import jax
import jax.numpy as jnp
from jax import lax
from jax.experimental import pallas as pl
from jax.experimental.pallas import tpu as pltpu
from jax.experimental.pallas import tpu_sc as plsc

N_TOKENS = 32768
D_MODEL = 768
N_EXPERTS = 64
CAPACITY = 512
GATE_H = 64
EXP_H = 128
PAD_ROWS = 2048
FLAT_ROWS = N_EXPERTS * CAPACITY + PAD_ROWS

TB = 1024
N_TBLOCKS = N_TOKENS // TB
D_HALF = D_MODEL // 2


def _pack_bf16(x32):
    lo = lax.bitcast_convert_type(x32[:, :D_HALF].astype(jnp.bfloat16),
                                  jnp.uint16).astype(jnp.uint32)
    hi = lax.bitcast_convert_type(x32[:, D_HALF:].astype(jnp.bfloat16),
                                  jnp.uint16).astype(jnp.uint32)
    return lax.bitcast_convert_type(lo | (hi << 16), jnp.float32)


def _unpack_bf16(p32):
    u = lax.bitcast_convert_type(p32, jnp.uint32)
    lo = lax.bitcast_convert_type((u & 0xFFFF).astype(jnp.uint16),
                                  jnp.bfloat16)
    hi = lax.bitcast_convert_type((u >> 16).astype(jnp.uint16), jnp.bfloat16)
    return jnp.concatenate([lo, hi], axis=1)

NC, NS = 2, 16
NW = NC * NS
TOK_PER_W = N_TOKENS // NW
K_CHUNK = 64
N_CHUNKS = TOK_PER_W // K_CHUNK
N_BUF = 4
N_ROUNDS = N_CHUNKS // N_BUF


def _gate_body(x_ref, wg1_ref, bg1_ref, wg2_ref, bg2_ref, sslot_ref, stok_ref,
               keep_ref, kcnt_ref, xbf_ref, counts_ref):
    i = pl.program_id(0)

    @pl.when(i == 0)
    def _():
        counts_ref[...] = jnp.zeros_like(counts_ref)

    x = x_ref[...]
    xbf_ref[...] = _pack_bf16(x)
    gh = jnp.dot(x, wg1_ref[...], preferred_element_type=jnp.float32)
    gh = jnp.maximum(gh + bg1_ref[...], 0.0)
    logits = jnp.dot(gh, wg2_ref[...], preferred_element_type=jnp.float32)
    logits = logits + bg2_ref[...]

    m = jnp.max(logits, axis=1, keepdims=True)
    iota_e = lax.broadcasted_iota(jnp.int32, (TB, N_EXPERTS), 1)
    sel = logits == m
    expert = jnp.min(jnp.where(sel, iota_e, N_EXPERTS), axis=1)
    first = expert[:, None] == iota_e

    ii = lax.broadcasted_iota(jnp.int32, (TB, TB), 0)
    jj = lax.broadcasted_iota(jnp.int32, (TB, TB), 1)
    tri = (jj <= ii).astype(jnp.bfloat16)
    onehot = first.astype(jnp.bfloat16)
    pos_incl = jnp.dot(tri, onehot, preferred_element_type=jnp.float32)

    prev = counts_ref[...]
    pos_all = pos_incl + prev
    pos_tok = jnp.sum(jnp.where(first, pos_all, 0.0), axis=1) - 1.0
    keep = pos_tok < CAPACITY
    slot = jnp.where(
        keep,
        expert * CAPACITY + pos_tok.astype(jnp.int32),
        N_EXPERTS * CAPACITY,
    )
    counts_ref[...] = prev + jnp.sum(first.astype(jnp.float32), axis=0,
                                     keepdims=True)

    keep_f = keep.astype(jnp.float32)
    kept_first = jnp.where(first, keep_f[:, None], 0.0)
    kcnt = jnp.sum(kept_first, axis=0, keepdims=True)
    e_a = lax.broadcasted_iota(jnp.int32, (N_EXPERTS, N_EXPERTS), 0)
    e_b = lax.broadcasted_iota(jnp.int32, (N_EXPERTS, N_EXPERTS), 1)
    t64 = (e_a < e_b).astype(jnp.bfloat16)
    prefix_e = (jnp.dot((kcnt // 256).astype(jnp.bfloat16), t64,
                        preferred_element_type=jnp.float32) * 256.0
                + jnp.dot((kcnt % 256).astype(jnp.bfloat16), t64,
                          preferred_element_type=jnp.float32))
    own_pos = jnp.sum(jnp.where(first, pos_incl, 0.0), axis=1) - 1.0
    pref_at_e = jnp.sum(jnp.where(first, prefix_e, 0.0), axis=1)
    drop_bf = (1.0 - keep_f).astype(jnp.bfloat16)
    dincl = jnp.dot(tri, drop_bf[:, None],
                    preferred_element_type=jnp.float32)[:, 0]
    total_kept = jnp.sum(keep_f)
    rank = jnp.where(keep, pref_at_e + own_pos,
                     total_kept + dincl - 1.0).astype(jnp.int32)

    CT = 256
    li = lax.broadcasted_iota(jnp.int32, (TB,), 0)
    vals = jnp.concatenate([
        (slot // 256).astype(jnp.bfloat16)[None, :],
        (slot % 256).astype(jnp.bfloat16)[None, :],
        (li // 256).astype(jnp.bfloat16)[None, :],
        (li % 256).astype(jnp.bfloat16)[None, :],
    ], axis=0)
    srt_parts = []
    for t in range(TB // CT):
        cj = lax.broadcasted_iota(jnp.int32, (TB, CT), 1) + t * CT
        pt = (rank[:, None] == cj).astype(jnp.bfloat16)
        srt_parts.append(jnp.dot(vals, pt, preferred_element_type=jnp.float32))
    srt = jnp.concatenate(srt_parts, axis=1)
    sslot_ref[...] = (srt[0] * 256.0 + srt[1]).astype(jnp.int32)[None, None, :]
    stok_ref[...] = ((srt[2] * 256.0 + srt[3]).astype(jnp.int32)
                     + i * TB)[None, None, :]
    keep_ref[...] = keep.astype(jnp.int32)[None, None, :]
    kcnt_ref[...] = jnp.full((1, 1, 128), total_kept.astype(jnp.int32),
                             jnp.int32)


def _gate_route(x, wg1, bg1, wg2, bg2):
    return pl.pallas_call(
        _gate_body,
        grid=(N_TBLOCKS,),
        in_specs=[
            pl.BlockSpec((TB, D_MODEL), lambda i: (i, 0)),
            pl.BlockSpec((D_MODEL, GATE_H), lambda i: (0, 0)),
            pl.BlockSpec((1, GATE_H), lambda i: (0, 0)),
            pl.BlockSpec((GATE_H, N_EXPERTS), lambda i: (0, 0)),
            pl.BlockSpec((1, N_EXPERTS), lambda i: (0, 0)),
        ],
        out_specs=[
            pl.BlockSpec((1, 1, TB), lambda i: (i, 0, 0)),
            pl.BlockSpec((1, 1, TB), lambda i: (i, 0, 0)),
            pl.BlockSpec((1, 1, TB), lambda i: (i, 0, 0)),
            pl.BlockSpec((1, 1, 128), lambda i: (i, 0, 0)),
            pl.BlockSpec((TB, D_HALF), lambda i: (i, 0)),
        ],
        out_shape=[
            jax.ShapeDtypeStruct((N_TBLOCKS, 1, TB), jnp.int32),
            jax.ShapeDtypeStruct((N_TBLOCKS, 1, TB), jnp.int32),
            jax.ShapeDtypeStruct((N_TBLOCKS, 1, TB), jnp.int32),
            jax.ShapeDtypeStruct((N_TBLOCKS, 1, 128), jnp.int32),
            jax.ShapeDtypeStruct((N_TOKENS, D_HALF), jnp.float32),
        ],
        scratch_shapes=[pltpu.VMEM((1, N_EXPERTS), jnp.float32)],
    )(x, wg1, bg1, wg2, bg2)


def _expert_body(disp_ref, w1_ref, b1_ref, w2_ref, b2_ref, out_ref):
    e = pl.program_id(0)

    @pl.when(e < N_EXPERTS)
    def _():
        xb = _unpack_bf16(disp_ref[...])
        h = jnp.dot(xb, w1_ref[0].astype(jnp.bfloat16),
                    preferred_element_type=jnp.float32)
        h = jnp.maximum(h + b1_ref[0], 0.0)
        z = jnp.dot(h.astype(jnp.bfloat16), w2_ref[0].astype(jnp.bfloat16),
                    preferred_element_type=jnp.float32)
        z = z + b2_ref[0]
        out_ref[...] = _pack_bf16(jax.nn.softmax(z, axis=-1))

    @pl.when(e == N_EXPERTS)
    def _():
        out_ref[...] = jnp.zeros_like(out_ref)


def _experts(disp, w1, b1, w2, b2):
    clamp = lambda e: jnp.minimum(e, N_EXPERTS - 1)
    return pl.pallas_call(
        _expert_body,
        grid=(N_EXPERTS + 1,),
        in_specs=[
            pl.BlockSpec((CAPACITY, D_HALF), lambda e: (clamp(e), 0)),
            pl.BlockSpec((1, D_MODEL, EXP_H), lambda e: (clamp(e), 0, 0)),
            pl.BlockSpec((1, 1, EXP_H), lambda e: (clamp(e), 0, 0)),
            pl.BlockSpec((1, EXP_H, D_MODEL), lambda e: (clamp(e), 0, 0)),
            pl.BlockSpec((1, 1, D_MODEL), lambda e: (clamp(e), 0, 0)),
        ],
        out_specs=pl.BlockSpec((CAPACITY, D_HALF), lambda e: (e, 0)),
        out_shape=jax.ShapeDtypeStruct((FLAT_ROWS, D_HALF), jnp.float32),
    )(disp, w1, b1.reshape(N_EXPERTS, 1, EXP_H), w2,
      b2.reshape(N_EXPERTS, 1, D_MODEL))


def _worker_id():
    return lax.axis_index("s") * NC + lax.axis_index("c")


@functools.cache
def _sc_kernels():
    mesh = plsc.VectorSubcoreMesh(core_axis_name="c", subcore_axis_name="s",
                                  num_cores=NC, num_subcores=NS)
    scratch = [
        pltpu.VMEM((N_CHUNKS, K_CHUNK), jnp.int32),
        pltpu.VMEM((N_CHUNKS, K_CHUNK), jnp.int32),
        pltpu.VMEM((K_CHUNK, D_HALF), jnp.float32),
        pltpu.VMEM((K_CHUNK, D_HALF), jnp.float32),
        pltpu.SemaphoreType.DMA,
        pltpu.SemaphoreType.DMA,
    ]

    def make_permute(n_out_rows):
        @functools.partial(
            pl.kernel,
            mesh=mesh,
            out_type=jax.ShapeDtypeStruct((n_out_rows, D_HALF), jnp.float32),
            scratch_types=scratch,
        )
        def permute(src_hbm, sidx_hbm, didx_hbm, out_hbm,
                    sidx_v, didx_v, rows0, rows1, sem0, sem1):
            rows, sems = (rows0, rows1), (sem0, sem1)
            wid = _worker_id()
            pltpu.sync_copy(sidx_hbm.at[wid], sidx_v)
            pltpu.sync_copy(didx_hbm.at[wid], didx_v)
            g = [None, None]
            s = [None, None]
            g[0] = pltpu.async_copy(src_hbm.at[sidx_v.at[0]], rows0, sem0)
            for j in range(N_CHUNKS):
                b = j % 2
                g[b].wait()
                if j + 1 < N_CHUNKS:
                    nb = 1 - b
                    if s[nb] is not None:
                        s[nb].wait()
                    g[nb] = pltpu.async_copy(
                        src_hbm.at[sidx_v.at[j + 1]], rows[nb], sems[nb])
                s[b] = pltpu.async_copy(rows[b], out_hbm.at[didx_v.at[j]],
                                        sems[b])
            s[0].wait()
            s[1].wait()

        return permute

    return make_permute(FLAT_ROWS), make_permute(FLAT_ROWS)


def _unpack_body(p_ref, keep_ref, out_ref):
    full = _unpack_bf16(p_ref[...]).astype(jnp.float32)
    out_ref[...] = jnp.where(keep_ref[0, 0][:, None] > 0, full, 0.0)


def _final_unpack(packed, keep):
    return pl.pallas_call(
        _unpack_body,
        grid=(N_TBLOCKS,),
        in_specs=[
            pl.BlockSpec((TB, D_HALF), lambda i: (i, 0)),
            pl.BlockSpec((1, 1, TB), lambda i: (i, 0, 0)),
        ],
        out_specs=pl.BlockSpec((TB, D_MODEL), lambda i: (i, 0)),
        out_shape=jax.ShapeDtypeStruct((N_TOKENS, D_MODEL), jnp.float32),
    )(packed, keep)


def _interleave(a3):
    return (a3.reshape(16, 2, N_CHUNKS, K_CHUNK)
            .transpose(1, 2, 0, 3).reshape(NW, N_CHUNKS, K_CHUNK))


def kernel(inputs, Wg1, bg1, Wg2, bg2, W1, b1, W2, b2):
    sslot, stok, keep, kcnt, xbf = _gate_route(
        inputs, Wg1, bg1.reshape(1, -1), Wg2, bg2.reshape(1, -1))
    sslotW = _interleave(sslot.reshape(N_TBLOCKS, N_CHUNKS, K_CHUNK))
    stokW = _interleave(stok.reshape(N_TBLOCKS, N_CHUNKS, K_CHUNK))
    cnt = kcnt.reshape(N_TBLOCKS, 128)[:, 0]
    w = jnp.arange(NW)[:, None]
    k = jnp.arange(N_CHUNKS)[None, :]
    live = ((w % 16) * K_CHUNK < cnt[2 * k + w // 16])[:, :, None]
    lane = jnp.arange(K_CHUNK)[None, None, :]
    src_skip = (w[:, :, None] * N_CHUNKS + k[:, :, None]) * K_CHUNK + lane
    dst_skip = N_TOKENS + w[:, :, None] * K_CHUNK + lane
    d_src = jnp.where(live, stokW, src_skip)
    d_dst = jnp.where(live, sslotW, dst_skip)
    c_src = jnp.where(live, sslotW, src_skip)
    c_dst = jnp.where(live, stokW, dst_skip)
    dispatch, combine = _sc_kernels()
    disp = dispatch(xbf, d_src, d_dst)
    flat = _experts(disp, W1, b1, W2, b2)
    return _final_unpack(combine(flat, c_src, c_dst), keep)
